# baseline (speedup 1.0000x reference)
import jax
import jax.numpy as jnp
from jax.experimental import pallas as pl
from jax.experimental.pallas import tpu as pltpu

_N_UNITS = 4096
_K = 410
_BOOST_STRENGTH = 1.0
_TARGET_DENSITY = float(_K) / _N_UNITS
_ROWS = 256

_INT_MIN = -2147483648


def _kwinners_block(x_ref, dc_ref, o_ref):
    xb = x_ref[...]
    dc = dc_ref[...]
    bf = jnp.exp((_TARGET_DENSITY - dc) * _BOOST_STRENGTH)
    boosted = xb * bf

    t = jax.lax.bitcast_convert_type(boosted, jnp.int32)
    key = t ^ ((t >> 31) & jnp.int32(0x7FFFFFFF))

    imin = jnp.int32(_INT_MIN)
    T = jnp.zeros((xb.shape[0], 1), jnp.int32)
    for b in range(31, -1, -1):
        bit = imin if b == 31 else jnp.int32(1 << b)
        trial = T | bit
        thr = trial ^ imin
        cnt = jnp.sum(key >= thr, axis=1, keepdims=True, dtype=jnp.int32)
        T = jnp.where(cnt >= _K, trial, T)

    kth = T ^ imin
    o_ref[...] = jnp.where(key >= kth, xb, 0.0)


def _kwinners_call(x, dc):
    B, N = x.shape
    return pl.pallas_call(
        _kwinners_block,
        grid=(B // _ROWS,),
        in_specs=[
            pl.BlockSpec((_ROWS, N), lambda i: (i, 0)),
            pl.BlockSpec((1, N), lambda i: (0, 0)),
        ],
        out_specs=pl.BlockSpec((_ROWS, N), lambda i: (i, 0)),
        out_shape=jax.ShapeDtypeStruct((B, N), x.dtype),
        compiler_params=pltpu.CompilerParams(
            dimension_semantics=("parallel",),
        ),
    )(x, dc)


def kernel(x, dutyCycle):
    B, N = x.shape
    dc = dutyCycle.reshape(1, N)
    devs = jax.devices()
    n_dev = 2 if len(devs) >= 2 and B % (2 * _ROWS) == 0 else 1
    if n_dev == 1:
        return _kwinners_call(x, dc)
    import numpy as np
    from jax.sharding import Mesh, PartitionSpec as P
    mesh = Mesh(np.array(devs[:n_dev]), ("b",))
    f = jax.shard_map(
        _kwinners_call,
        mesh=mesh,
        in_specs=(P("b", None), P(None, None)),
        out_specs=P("b", None),
        check_vma=False,
    )
    return f(x, dc)


# speedup vs baseline: 2.0333x; 2.0333x over previous
import jax
import jax.numpy as jnp
from jax.experimental import pallas as pl
from jax.experimental.pallas import tpu as pltpu

_N_UNITS = 4096
_K = 410
_BOOST_STRENGTH = 1.0
_TARGET_DENSITY = float(_K) / _N_UNITS
_ROWS = 256


def _bisect16(vals16, quota):
    R = vals16.shape[0]
    T = jnp.zeros((R, 1), jnp.int32)
    for b in range(15, -1, -1):
        trial = T | jnp.int32(1 << b)
        thr16 = (trial - jnp.int32(0x8000)).astype(jnp.int16)
        s = (vals16 >= thr16).astype(jnp.int16)
        w = s.shape[1]
        while w > 128:
            w //= 2
            s = s[:, :w] + s[:, w:]
        cnt = jnp.sum(s.astype(jnp.int32), axis=1, keepdims=True)
        T = jnp.where(cnt >= quota, trial, T)
    return T - jnp.int32(0x8000)


def _kwinners_block(x_ref, dc_ref, o_ref):
    xb = x_ref[...]
    dc = dc_ref[...]
    bf = jnp.exp((_TARGET_DENSITY - dc) * _BOOST_STRENGTH)
    boosted = xb * bf

    t = jax.lax.bitcast_convert_type(boosted, jnp.int32)
    key = t ^ ((t >> 31) & jnp.int32(0x7FFFFFFF))

    R = xb.shape[0]

    k_hi = (key >> 16).astype(jnp.int16)
    k_lo = ((key & jnp.int32(0xFFFF)) - jnp.int32(0x8000)).astype(jnp.int16)

    quota = jnp.full((R, 1), _K, jnp.int32)

    t_hi = _bisect16(k_hi, quota)
    t_hi16 = t_hi.astype(jnp.int16)

    sgt = (k_hi > t_hi16).astype(jnp.int16)
    w = sgt.shape[1]
    while w > 128:
        w //= 2
        sgt = sgt[:, :w] + sgt[:, w:]
    cnt_gt = jnp.sum(sgt.astype(jnp.int32), axis=1, keepdims=True)
    k_rem = quota - cnt_gt

    eq = k_hi == t_hi16
    ml = jnp.where(eq, k_lo, jnp.int16(-32768))

    t_lo = _bisect16(ml, k_rem)
    t_lo16 = t_lo.astype(jnp.int16)

    win = (k_hi > t_hi16) | (eq & (ml >= t_lo16))
    o_ref[...] = jnp.where(win, xb, 0.0)


def kernel(x, dutyCycle):
    B, N = x.shape
    dc = dutyCycle.reshape(1, N)
    return pl.pallas_call(
        _kwinners_block,
        grid=(B // _ROWS,),
        in_specs=[
            pl.BlockSpec((_ROWS, N), lambda i: (i, 0)),
            pl.BlockSpec((1, N), lambda i: (0, 0)),
        ],
        out_specs=pl.BlockSpec((_ROWS, N), lambda i: (i, 0)),
        out_shape=jax.ShapeDtypeStruct((B, N), x.dtype),
        compiler_params=pltpu.CompilerParams(
            dimension_semantics=("parallel",),
        ),
    )(x, dc)
